# baseline (speedup 1.0000x reference)
import jax
import jax.numpy as jnp
from jax import lax
from jax.experimental import pallas as pl
from jax.experimental.pallas import tpu as pltpu

B = 4096
D = 1024
N = 8
C = 1000
CP = 1024
BN_EPS = 1e-3
LOG_EPS = 1e-30
SBLK = 1024
ABLK = 2048
NSB = B // SBLK
NAB = B // ABLK
GRID = NSB + NAB


def _body(h_ref, m_ref, lab_ref, W_ref, b_ref, g_ref, be_ref,
          outh_ref, outig_ref, outr_ref,
          s1, s2, cnt, pnc, hbuf):
    g = pl.program_id(0)
    i = g
    j = g - NSB

    @pl.when(g == 0)
    def _init():
        s1[...] = jnp.zeros_like(s1)
        s2[...] = jnp.zeros_like(s2)
        cnt[...] = jnp.zeros_like(cnt)
        pnc[...] = jnp.zeros_like(pnc)

    @pl.when(g < NSB)
    def _stats():
        x = h_ref[...]
        hbuf[pl.ds(i * SBLK, SBLK), :] = x
        w = m_ref[pl.ds(i * SBLK, SBLK), :]
        wx = x * w
        s1[...] += jnp.sum(wx, axis=0, keepdims=True)
        s2[...] += jnp.sum(wx * x, axis=0, keepdims=True)
        cnt[...] += jnp.sum(w, axis=0, keepdims=True)

    @pl.when(g == NSB)
    def _finalize_stats():
        denom = cnt[...] + 1e-8
        mean = s1[...] / denom
        var = s2[...] / denom - mean * mean
        scale = lax.rsqrt(var + BN_EPS) * g_ref[...]
        s1[...] = scale
        s2[...] = be_ref[...] - mean * scale

    @pl.when(g >= NSB)
    def _apply():
        x = hbuf[pl.ds(j * ABLK, ABLK), :]
        xn = x * s1[...] + s2[...]
        outh_ref[...] = xn
        act = jnp.dot(xn, W_ref[...], preferred_element_type=jnp.float32)
        act = act + b_ref[...]
        mx = jnp.max(act, axis=1, keepdims=True)
        e = jnp.exp(act - mx)
        p = e / jnp.sum(e, axis=1, keepdims=True)
        w = m_ref[pl.ds(j * ABLK, ABLK), :]
        wp = p * w
        lab = lab_ref[pl.ds(j * ABLK, ABLK), :]
        iota_c = lax.broadcasted_iota(jnp.int32, (ABLK, CP), 1)
        onehot = (iota_c == lab).astype(jnp.bfloat16)
        pnc[...] += lax.dot_general(
            wp.astype(jnp.bfloat16), onehot, (((0,), (0,)), ((), ())),
            preferred_element_type=jnp.float32)
        iota_n = lax.broadcasted_iota(jnp.int32, (ABLK, N), 1)
        big = jnp.where(act == mx, iota_n, N)
        amin = jnp.min(big, axis=1, keepdims=True)
        outr_ref[...] = ((iota_n == amin) & (w > 0.5)).astype(jnp.int32)

    @pl.when(g == GRID - 1)
    def _entropy():
        denom = cnt[...] + 1e-8
        pcn = pnc[...] / denom
        pn = jnp.sum(pcn, axis=1, keepdims=True)
        pc = jnp.sum(pcn, axis=0, keepdims=True)
        ent_cn = -jnp.sum(pcn * jnp.log(pcn + LOG_EPS))
        ent_n = -jnp.sum(pn * jnp.log(pn + LOG_EPS))
        ent_c = -jnp.sum(pc * jnp.log(pc + LOG_EPS))
        outig_ref[...] = jnp.full((1, 1), -(ent_n + ent_c - ent_cn),
                                  dtype=jnp.float32)


@jax.jit
def kernel(h_net, ig_mask, labels, W, b, gamma, beta):
    mask_f = ig_mask.astype(jnp.float32).reshape(B, 1)
    lab = labels.astype(jnp.int32).reshape(B, 1)
    outs = pl.pallas_call(
        _body,
        grid=(GRID,),
        in_specs=[
            pl.BlockSpec((SBLK, D), lambda g: (jnp.where(g < NSB, g, NSB - 1), 0)),
            pl.BlockSpec((B, 1), lambda g: (0, 0)),
            pl.BlockSpec((B, 1), lambda g: (0, 0)),
            pl.BlockSpec((D, N), lambda g: (0, 0)),
            pl.BlockSpec((1, N), lambda g: (0, 0)),
            pl.BlockSpec((1, D), lambda g: (0, 0)),
            pl.BlockSpec((1, D), lambda g: (0, 0)),
        ],
        out_specs=[
            pl.BlockSpec((ABLK, D), lambda g: (jnp.where(g < NSB, 0, g - NSB), 0)),
            pl.BlockSpec((1, 1), lambda g: (0, 0)),
            pl.BlockSpec((ABLK, N), lambda g: (jnp.where(g < NSB, 0, g - NSB), 0)),
        ],
        out_shape=[
            jax.ShapeDtypeStruct((B, D), jnp.float32),
            jax.ShapeDtypeStruct((1, 1), jnp.float32),
            jax.ShapeDtypeStruct((B, N), jnp.int32),
        ],
        scratch_shapes=[
            pltpu.VMEM((1, D), jnp.float32),
            pltpu.VMEM((1, D), jnp.float32),
            pltpu.VMEM((1, 1), jnp.float32),
            pltpu.VMEM((N, CP), jnp.float32),
            pltpu.VMEM((B, D), jnp.float32),
        ],
    )(h_net, mask_f, lab, W, b.reshape(1, N), gamma.reshape(1, D),
      beta.reshape(1, D))
    h_normed, ig, routing = outs
    return h_normed, ig[0, 0], routing


# speedup vs baseline: 1.0450x; 1.0450x over previous
import jax
import jax.numpy as jnp
from jax import lax
from jax.experimental import pallas as pl
from jax.experimental.pallas import tpu as pltpu

B = 4096
D = 1024
N = 8
C = 1000
CP = 1024
BN_EPS = 1e-3
LOG_EPS = 1e-30
SBLK = 2048
ABLK = 1024
NSB = B // SBLK
NAB = B // ABLK
GRID = NSB + NAB


def _body(h_ref, m_ref, lab_ref, W_ref, b_ref, g_ref, be_ref,
          outh_ref, outig_ref, outr_ref,
          s1, s2, cnt, pnc, hbuf):
    g = pl.program_id(0)
    i = g
    j = g - NSB

    @pl.when(g == 0)
    def _init():
        s1[...] = jnp.zeros_like(s1)
        s2[...] = jnp.zeros_like(s2)
        cnt[...] = jnp.zeros_like(cnt)
        pnc[...] = jnp.zeros_like(pnc)

    @pl.when(g < NSB)
    def _stats():
        x = h_ref[...]
        hbuf[pl.ds(i * SBLK, SBLK), :] = x
        w = m_ref[pl.ds(i * SBLK, SBLK), :]
        wx = x * w
        s1[...] += jnp.sum(wx, axis=0, keepdims=True)
        s2[...] += jnp.sum(wx * x, axis=0, keepdims=True)
        cnt[...] += jnp.sum(w, axis=0, keepdims=True)

    @pl.when(g == NSB)
    def _finalize_stats():
        denom = cnt[...] + 1e-8
        mean = s1[...] / denom
        var = s2[...] / denom - mean * mean
        scale = lax.rsqrt(var + BN_EPS) * g_ref[...]
        s1[...] = scale
        s2[...] = be_ref[...] - mean * scale

    @pl.when(g >= NSB)
    def _apply():
        x = hbuf[pl.ds(j * ABLK, ABLK), :]
        xn = x * s1[...] + s2[...]
        outh_ref[...] = xn
        act = jnp.dot(xn, W_ref[...], preferred_element_type=jnp.float32)
        act = act + b_ref[...]
        mx = jnp.max(act, axis=1, keepdims=True)
        e = jnp.exp(act - mx)
        p = e / jnp.sum(e, axis=1, keepdims=True)
        w = m_ref[pl.ds(j * ABLK, ABLK), :]
        wp = p * w
        lab = lab_ref[pl.ds(j * ABLK, ABLK), :]
        iota_c = lax.broadcasted_iota(jnp.int32, (ABLK, CP), 1)
        onehot = (iota_c == lab).astype(jnp.bfloat16)
        pnc[...] += lax.dot_general(
            wp.astype(jnp.bfloat16), onehot, (((0,), (0,)), ((), ())),
            preferred_element_type=jnp.float32)
        iota_n = lax.broadcasted_iota(jnp.int32, (ABLK, N), 1)
        big = jnp.where(act == mx, iota_n, N)
        amin = jnp.min(big, axis=1, keepdims=True)
        outr_ref[...] = ((iota_n == amin) & (w > 0.5)).astype(jnp.int32)

    @pl.when(g == GRID - 1)
    def _entropy():
        denom = cnt[...] + 1e-8
        pcn = pnc[...] / denom
        pn = jnp.sum(pcn, axis=1, keepdims=True)
        pc = jnp.sum(pcn, axis=0, keepdims=True)
        ent_cn = -jnp.sum(pcn * jnp.log(pcn + LOG_EPS))
        ent_n = -jnp.sum(pn * jnp.log(pn + LOG_EPS))
        ent_c = -jnp.sum(pc * jnp.log(pc + LOG_EPS))
        outig_ref[...] = jnp.full((1, 1), -(ent_n + ent_c - ent_cn),
                                  dtype=jnp.float32)


@jax.jit
def kernel(h_net, ig_mask, labels, W, b, gamma, beta):
    mask_f = ig_mask.astype(jnp.float32).reshape(B, 1)
    lab = labels.astype(jnp.int32).reshape(B, 1)
    outs = pl.pallas_call(
        _body,
        grid=(GRID,),
        in_specs=[
            pl.BlockSpec((SBLK, D), lambda g: (jnp.where(g < NSB, g, NSB - 1), 0)),
            pl.BlockSpec((B, 1), lambda g: (0, 0)),
            pl.BlockSpec((B, 1), lambda g: (0, 0)),
            pl.BlockSpec((D, N), lambda g: (0, 0)),
            pl.BlockSpec((1, N), lambda g: (0, 0)),
            pl.BlockSpec((1, D), lambda g: (0, 0)),
            pl.BlockSpec((1, D), lambda g: (0, 0)),
        ],
        out_specs=[
            pl.BlockSpec((ABLK, D), lambda g: (jnp.where(g < NSB, 0, g - NSB), 0)),
            pl.BlockSpec((1, 1), lambda g: (0, 0)),
            pl.BlockSpec((ABLK, N), lambda g: (jnp.where(g < NSB, 0, g - NSB), 0)),
        ],
        out_shape=[
            jax.ShapeDtypeStruct((B, D), jnp.float32),
            jax.ShapeDtypeStruct((1, 1), jnp.float32),
            jax.ShapeDtypeStruct((B, N), jnp.int32),
        ],
        scratch_shapes=[
            pltpu.VMEM((1, D), jnp.float32),
            pltpu.VMEM((1, D), jnp.float32),
            pltpu.VMEM((1, 1), jnp.float32),
            pltpu.VMEM((N, CP), jnp.float32),
            pltpu.VMEM((B, D), jnp.float32),
        ],
    )(h_net, mask_f, lab, W, b.reshape(1, N), gamma.reshape(1, D),
      beta.reshape(1, D))
    h_normed, ig, routing = outs
    return h_normed, ig[0, 0], routing
